# baseline (speedup 1.0000x reference)
import functools
import math

import jax
import jax.numpy as jnp
from jax import lax
from jax.experimental import pallas as pl
from jax.experimental.pallas import tpu as pltpu
from jax.experimental.pallas import tpu_sc as plsc

EMB_DIM = 64
SCALE = math.sqrt(EMB_DIM)
LANES = 16
CSTRIDE = 133
NGBUF = 4
NTBUF = 2


def _make_sc_gather(b_d0: int, s_d1: int, vocab: int, d: int, dpad: int):
  info = plsc.get_sparse_core_info()
  nw = info.num_cores * info.num_subcores
  assert b_d0 == nw * 128
  n_units = s_d1
  assert n_units % 2 == 0
  n_half = 2 * n_units

  mesh = plsc.VectorSubcoreMesh(core_axis_name="c", subcore_axis_name="s")

  @functools.partial(
      pl.kernel,
      mesh=mesh,
      out_type=jax.ShapeDtypeStruct((s_d1, d // 8, nw, 8, 128), jnp.float32),
      scratch_types=[
          pltpu.VMEM((s_d1, 128), jnp.int32),
      ]
      + [pltpu.VMEM((64, dpad), jnp.float32)] * NGBUF
      + [pltpu.VMEM((d // 8, 8, CSTRIDE), jnp.float32)] * NTBUF
      + [pltpu.SemaphoreType.DMA] * (NGBUF + NTBUF),
      compiler_params=pltpu.CompilerParams(needs_layout_passes=False),
  )
  def gather_kernel(idx_hbm, table_hbm, out_hbm, idx_v, *bufs_and_sems):
    gbuf = bufs_and_sems[:NGBUF]
    tbuf = bufs_and_sems[NGBUF : NGBUF + NTBUF]
    gin = bufs_and_sems[NGBUF + NTBUF : 2 * NGBUF + NTBUF]
    gout = bufs_and_sems[2 * NGBUF + NTBUF :]
    wid = lax.axis_index("s") * info.num_cores + lax.axis_index("c")
    pltpu.sync_copy(idx_hbm.at[:, pl.ds(wid * 128, 128)], idx_v)

    def start_gather(t, g):
      pltpu.async_copy(
          table_hbm.at[idx_v.at[t // 2, pl.ds((t % 2) * 64, 64)]],
          gbuf[g],
          gin[g],
      )

    def wait_gather(g):
      pltpu.make_async_copy(table_hbm.at[pl.ds(0, 64)], gbuf[g], gin[g]).wait()

    def start_out(j, tb):
      pltpu.async_copy(
          tbuf[tb].at[:, :, pl.ds(0, 128)], out_hbm.at[j, :, wid], gout[tb]
      )

    def wait_out(tb):
      pltpu.make_async_copy(
          tbuf[tb].at[:, :, pl.ds(0, 128)], out_hbm.at[0, :, wid], gout[tb]
      ).wait()

    t2_ids, s_ids = [], []
    for v in range(d // LANES):
      r = lax.iota(jnp.int32, 16) + v * LANES
      t2_ids.append(lax.shift_right_logical(r, 3))
      s_ids.append(lax.bitwise_and(r, 7))

    start_gather(0, 0)
    start_gather(1, 1)

    def body(i, carry):
      for jj in range(2):
        j = 2 * i + jj
        tb = jj

        @pl.when(j >= 2)
        def _():
          wait_out(tb)

        for h in range(2):
          t = 2 * j + h
          g = (2 * jj + h) % NGBUF

          @pl.when(t + 2 < n_half)
          def _():
            start_gather(t + 2, (g + 2) % NGBUF)

          wait_gather(g)

          def trans_body(k, c2):
            col = jnp.full((16,), k + 64 * h, jnp.int32)
            for v in range(d // LANES):
              vals = gbuf[g][k, pl.ds(v * LANES, LANES)] * SCALE
              plsc.store_scatter(tbuf[tb], [t2_ids[v], s_ids[v], col], vals)
            return c2

          lax.fori_loop(0, 64, trans_body, 0, unroll=4)
        start_out(j, tb)
      return carry

    lax.fori_loop(0, n_units // 2, body, 0)
    for tb in range(NTBUF):
      wait_out(tb)

  return gather_kernel


@jax.jit
def kernel(tokens, table):
  b, s = tokens.shape
  vocab, d = table.shape
  dpad = 2 * d
  tokT = tokens.T
  table_p = jnp.pad(table, ((0, 0), (0, dpad - d)))
  out5 = _make_sc_gather(b, s, vocab, d, dpad)(tokT, table_p)
  return out5.transpose(2, 4, 0, 1, 3).reshape(b, s, d)


# speedup vs baseline: 1.0313x; 1.0313x over previous
import functools
import math

import jax
import jax.numpy as jnp
from jax import lax
from jax.experimental import pallas as pl
from jax.experimental.pallas import tpu as pltpu
from jax.experimental.pallas import tpu_sc as plsc

EMB_DIM = 64
SCALE = math.sqrt(EMB_DIM)
LANES = 16
CHUNK = 128
NBUF = 4


def _make_sc_gather(n_tokens: int, vocab: int, d: int, dpad: int):
  info = plsc.get_sparse_core_info()
  nw = info.num_cores * info.num_subcores
  assert n_tokens % (nw * CHUNK) == 0
  per_w = n_tokens // nw
  n_chunks = per_w // CHUNK
  assert n_chunks % NBUF == 0

  mesh = plsc.VectorSubcoreMesh(core_axis_name="c", subcore_axis_name="s")

  @functools.partial(
      pl.kernel,
      mesh=mesh,
      out_type=jax.ShapeDtypeStruct((n_tokens, dpad), jnp.float32),
      scratch_types=[
          pltpu.VMEM((per_w,), jnp.int32),
          pltpu.VMEM((per_w,), jnp.int32),
      ]
      + [pltpu.VMEM((CHUNK, dpad), jnp.float32)] * NBUF
      + [pltpu.SemaphoreType.DMA] * (2 * NBUF),
      compiler_params=pltpu.CompilerParams(needs_layout_passes=False),
  )
  def gather_kernel(idx_hbm, table_hbm, out_hbm, idx_v, pidx_v, *bufs_and_sems):
    rows = bufs_and_sems[:NBUF]
    gin = bufs_and_sems[NBUF : 2 * NBUF]
    gout = bufs_and_sems[2 * NBUF :]
    wid = lax.axis_index("s") * info.num_cores + lax.axis_index("c")
    base = wid * per_w
    pltpu.sync_copy(idx_hbm.at[pl.ds(base, per_w)], idx_v)

    def shift_body(k, carry):
      sl = pl.ds(k * LANES, LANES)
      pidx_v[sl] = lax.shift_right_logical(idx_v[sl], 1)
      return carry

    lax.fori_loop(0, per_w // LANES, shift_body, 0, unroll=4)

    def start_gather(j, b):
      pltpu.async_copy(
          table_hbm.at[pidx_v.at[pl.ds(j * CHUNK, CHUNK)]], rows[b], gin[b]
      )

    def wait_gather(b):
      pltpu.make_async_copy(table_hbm.at[pl.ds(0, CHUNK)], rows[b], gin[b]).wait()

    def start_out(j, b):
      pltpu.async_copy(
          rows[b], out_hbm.at[pl.ds(base + j * CHUNK, CHUNK)], gout[b]
      )

    def wait_out(b):
      pltpu.make_async_copy(
          rows[b], out_hbm.at[pl.ds(base, CHUNK)], gout[b]
      ).wait()

    for c in range(NBUF - 1):
      start_gather(c, c)

    def body(i, carry):
      for b in range(NBUF):
        j = i * NBUF + b
        bn = (b + NBUF - 1) % NBUF
        if b == 0:

          @pl.when(j + NBUF - 1 < n_chunks)
          def _():
            @pl.when(j >= 1)
            def _():
              wait_out(bn)

            start_gather(j + NBUF - 1, bn)
        else:

          @pl.when(j + NBUF - 1 < n_chunks)
          def _():
            wait_out(bn)
            start_gather(j + NBUF - 1, bn)

        wait_gather(b)

        def scale_body(r, c2):
          tok = plsc.load_gather(
              idx_v, [jnp.full((LANES,), j * CHUNK + r, jnp.int32)]
          )
          off = lax.bitwise_and(tok, 1) * d + lax.iota(jnp.int32, LANES)
          rvec = jnp.full((LANES,), r, jnp.int32)
          for v in range(d // LANES):
            src = plsc.load_gather(rows[b], [rvec, off + v * LANES])
            rows[b][r, pl.ds(v * LANES, LANES)] = src * SCALE
          return c2

        lax.fori_loop(0, CHUNK, scale_body, 0, unroll=4)
        start_out(j, b)
      return carry

    lax.fori_loop(0, n_chunks // NBUF, body, 0)
    for b in range(NBUF):
      wait_out(b)

  return gather_kernel


@jax.jit
def kernel(tokens, table):
  b, s = tokens.shape
  vocab, d = table.shape
  dpad = 2 * d
  n = b * s
  idx = tokens.reshape(n)
  table_pairs = table.reshape(vocab // 2, dpad)
  out = _make_sc_gather(n, vocab, d, dpad)(idx, table_pairs)
  return out[:, :d].reshape(b, s, d)


# speedup vs baseline: 1.6540x; 1.6038x over previous
import functools
import math

import jax
import jax.numpy as jnp
from jax import lax
from jax.experimental import pallas as pl
from jax.experimental.pallas import tpu as pltpu
from jax.experimental.pallas import tpu_sc as plsc

EMB_DIM = 64
SCALE = math.sqrt(EMB_DIM)
LANES = 16
CHUNK = 160
NBUF = 4
TCN = 2048


def _tc_stage(d: int, vocab: int, dpad: int):
  def body(tt_ref, out_ref):
    x = tt_ref[...]
    y = jnp.swapaxes(x, 0, 1) * SCALE
    out_ref[...] = jnp.concatenate([y, y], axis=1)

  return pl.pallas_call(
      body,
      grid=((vocab + TCN - 1) // TCN,),
      in_specs=[pl.BlockSpec((d, TCN), lambda i: (0, i))],
      out_specs=pl.BlockSpec((TCN, dpad), lambda i: (i, 0)),
      out_shape=jax.ShapeDtypeStruct((vocab, dpad), jnp.float32),
  )


def _make_sc_gather(n_tokens: int, vocab: int, d: int, dpad: int):
  info = plsc.get_sparse_core_info()
  nw = info.num_cores * info.num_subcores
  assert n_tokens % (nw * CHUNK) == 0
  per_w = n_tokens // nw
  n_chunks = per_w // CHUNK
  assert n_chunks % NBUF == 0

  mesh = plsc.VectorSubcoreMesh(core_axis_name="c", subcore_axis_name="s")

  @functools.partial(
      pl.kernel,
      mesh=mesh,
      out_type=jax.ShapeDtypeStruct((n_tokens, dpad), jnp.float32),
      scratch_types=[
          pltpu.VMEM((per_w,), jnp.int32),
      ]
      + [pltpu.VMEM((CHUNK, dpad), jnp.float32)] * NBUF
      + [pltpu.SemaphoreType.DMA] * (2 * NBUF),
  )
  def gather_kernel(idx_hbm, table_hbm, out_hbm, idx_v, *bufs_and_sems):
    rows = bufs_and_sems[:NBUF]
    gin = bufs_and_sems[NBUF : 2 * NBUF]
    gout = bufs_and_sems[2 * NBUF :]
    wid = lax.axis_index("s") * info.num_cores + lax.axis_index("c")
    base = wid * per_w
    pltpu.sync_copy(idx_hbm.at[pl.ds(base, per_w)], idx_v)

    def start_gather(j, b):
      pltpu.async_copy(
          table_hbm.at[idx_v.at[pl.ds(j * CHUNK, CHUNK)]], rows[b], gin[b]
      )

    def wait_gather(b):
      pltpu.make_async_copy(table_hbm.at[pl.ds(0, CHUNK)], rows[b], gin[b]).wait()

    def start_out(j, b):
      pltpu.async_copy(
          rows[b], out_hbm.at[pl.ds(base + j * CHUNK, CHUNK)], gout[b]
      )

    def wait_out(b):
      pltpu.make_async_copy(
          rows[b], out_hbm.at[pl.ds(base, CHUNK)], gout[b]
      ).wait()

    for c in range(NBUF - 1):
      start_gather(c, c)

    def body(i, carry):
      for b in range(NBUF):
        j = i * NBUF + b
        bn = (b + NBUF - 1) % NBUF
        if b == 0:

          @pl.when(j + NBUF - 1 < n_chunks)
          def _():
            @pl.when(j >= 1)
            def _():
              wait_out(bn)

            start_gather(j + NBUF - 1, bn)
        else:

          @pl.when(j + NBUF - 1 < n_chunks)
          def _():
            wait_out(bn)
            start_gather(j + NBUF - 1, bn)

        wait_gather(b)
        start_out(j, b)
      return carry

    lax.fori_loop(0, n_chunks // NBUF, body, 0)
    for b in range(NBUF):
      wait_out(b)

  return gather_kernel


@jax.jit
def kernel(tokens, table):
  b, s = tokens.shape
  vocab, d = table.shape
  dpad = 2 * d
  n = b * s
  idx = tokens.reshape(n)
  tableT = table.T
  table_p = _tc_stage(d, vocab, dpad)(tableT)
  out = _make_sc_gather(n, vocab, d, dpad)(idx, table_p)
  return out[:, :d].reshape(b, s, d)


# speedup vs baseline: 2.0510x; 1.2400x over previous
import functools
import math

import jax
import jax.numpy as jnp
from jax import lax
from jax.experimental import pallas as pl
from jax.experimental.pallas import tpu as pltpu
from jax.experimental.pallas import tpu_sc as plsc

EMB_DIM = 64
SCALE = math.sqrt(EMB_DIM)
LANES = 16
CHUNK = 160
NBUF = 4
TCN = 8192


def _tc_stage(d: int, vocab: int, dpad: int):
  def body(tt_ref, out_ref):
    x = tt_ref[...]
    y = jnp.swapaxes(x, 0, 1) * SCALE
    out_ref[...] = jnp.concatenate([y, y], axis=1)

  return pl.pallas_call(
      body,
      grid=((vocab + TCN - 1) // TCN,),
      in_specs=[pl.BlockSpec((d, TCN), lambda i: (0, i))],
      out_specs=pl.BlockSpec((TCN, dpad), lambda i: (i, 0)),
      out_shape=jax.ShapeDtypeStruct((vocab, dpad), jnp.float32),
  )


def _make_sc_gather(n_tokens: int, vocab: int, d: int, dpad: int):
  info = plsc.get_sparse_core_info()
  nw = info.num_cores * info.num_subcores
  assert n_tokens % (nw * CHUNK) == 0
  per_w = n_tokens // nw
  n_chunks = per_w // CHUNK
  assert n_chunks % NBUF == 0

  mesh = plsc.VectorSubcoreMesh(core_axis_name="c", subcore_axis_name="s")

  @functools.partial(
      pl.kernel,
      mesh=mesh,
      out_type=jax.ShapeDtypeStruct((n_tokens, dpad), jnp.float32),
      scratch_types=[
          pltpu.VMEM((per_w,), jnp.int32),
      ]
      + [pltpu.VMEM((CHUNK, dpad), jnp.float32)] * NBUF
      + [pltpu.SemaphoreType.DMA] * (2 * NBUF),
  )
  def gather_kernel(idx_hbm, table_hbm, out_hbm, idx_v, *bufs_and_sems):
    rows = bufs_and_sems[:NBUF]
    gin = bufs_and_sems[NBUF : 2 * NBUF]
    gout = bufs_and_sems[2 * NBUF :]
    wid = lax.axis_index("s") * info.num_cores + lax.axis_index("c")
    base = wid * per_w
    pltpu.sync_copy(idx_hbm.at[pl.ds(base, per_w)], idx_v)

    def start_gather(j, b):
      pltpu.async_copy(
          table_hbm.at[idx_v.at[pl.ds(j * CHUNK, CHUNK)]], rows[b], gin[b]
      )

    def wait_gather(b):
      pltpu.make_async_copy(table_hbm.at[pl.ds(0, CHUNK)], rows[b], gin[b]).wait()

    def start_out(j, b):
      pltpu.async_copy(
          rows[b], out_hbm.at[pl.ds(base + j * CHUNK, CHUNK)], gout[b]
      )

    def wait_out(b):
      pltpu.make_async_copy(
          rows[b], out_hbm.at[pl.ds(base, CHUNK)], gout[b]
      ).wait()

    for c in range(NBUF - 1):
      start_gather(c, c)

    def body(i, carry):
      for b in range(NBUF):
        j = i * NBUF + b
        bn = (b + NBUF - 1) % NBUF
        if b == 0:

          @pl.when(j + NBUF - 1 < n_chunks)
          def _():
            @pl.when(j >= 1)
            def _():
              wait_out(bn)

            start_gather(j + NBUF - 1, bn)
        else:

          @pl.when(j + NBUF - 1 < n_chunks)
          def _():
            wait_out(bn)
            start_gather(j + NBUF - 1, bn)

        wait_gather(b)
        start_out(j, b)
      return carry

    lax.fori_loop(0, n_chunks // NBUF, body, 0)
    for b in range(NBUF):
      wait_out(b)

  return gather_kernel


@jax.jit
def kernel(tokens, table):
  b, s = tokens.shape
  vocab, d = table.shape
  dpad = 2 * d
  n = b * s
  idx = tokens.reshape(n)
  tableT = table.T
  table_p = _tc_stage(d, vocab, dpad)(tableT)
  out = _make_sc_gather(n, vocab, d, dpad)(idx, table_p)
  return out[:, :d].reshape(b, s, d)


# speedup vs baseline: 2.1392x; 1.0430x over previous
import functools
import math

import jax
import jax.numpy as jnp
from jax import lax
from jax.experimental import pallas as pl
from jax.experimental.pallas import tpu as pltpu
from jax.experimental.pallas import tpu_sc as plsc

EMB_DIM = 64
SCALE = math.sqrt(EMB_DIM)
LANES = 16
CHUNK = 160
NBUF = 4
TCN = 16384


def _tc_stage(d: int, vocab: int, dpad: int):
  def body(tt_ref, out_ref):
    x = tt_ref[...]
    y = jnp.swapaxes(x, 0, 1) * SCALE
    out_ref[...] = jnp.concatenate([y, y], axis=1)

  return pl.pallas_call(
      body,
      grid=((vocab + TCN - 1) // TCN,),
      in_specs=[pl.BlockSpec((d, TCN), lambda i: (0, i))],
      out_specs=pl.BlockSpec((TCN, dpad), lambda i: (i, 0)),
      out_shape=jax.ShapeDtypeStruct((vocab, dpad), jnp.float32),
  )


def _make_sc_gather(n_tokens: int, vocab: int, d: int, dpad: int):
  info = plsc.get_sparse_core_info()
  nw = info.num_cores * info.num_subcores
  assert n_tokens % (nw * CHUNK) == 0
  per_w = n_tokens // nw
  n_chunks = per_w // CHUNK
  assert n_chunks % NBUF == 0

  mesh = plsc.VectorSubcoreMesh(core_axis_name="c", subcore_axis_name="s")

  @functools.partial(
      pl.kernel,
      mesh=mesh,
      out_type=jax.ShapeDtypeStruct((n_tokens, dpad), jnp.float32),
      scratch_types=[
          pltpu.VMEM((per_w,), jnp.int32),
      ]
      + [pltpu.VMEM((CHUNK, dpad), jnp.float32)] * NBUF
      + [pltpu.SemaphoreType.DMA] * (2 * NBUF),
  )
  def gather_kernel(idx_hbm, table_hbm, out_hbm, idx_v, *bufs_and_sems):
    rows = bufs_and_sems[:NBUF]
    gin = bufs_and_sems[NBUF : 2 * NBUF]
    gout = bufs_and_sems[2 * NBUF :]
    wid = lax.axis_index("s") * info.num_cores + lax.axis_index("c")
    base = wid * per_w
    pltpu.sync_copy(idx_hbm.at[pl.ds(base, per_w)], idx_v)

    def start_gather(j, b):
      pltpu.async_copy(
          table_hbm.at[idx_v.at[pl.ds(j * CHUNK, CHUNK)]], rows[b], gin[b]
      )

    def wait_gather(b):
      pltpu.make_async_copy(table_hbm.at[pl.ds(0, CHUNK)], rows[b], gin[b]).wait()

    def start_out(j, b):
      pltpu.async_copy(
          rows[b], out_hbm.at[pl.ds(base + j * CHUNK, CHUNK)], gout[b]
      )

    def wait_out(b):
      pltpu.make_async_copy(
          rows[b], out_hbm.at[pl.ds(base, CHUNK)], gout[b]
      ).wait()

    for c in range(NBUF - 1):
      start_gather(c, c)

    def body(i, carry):
      for b in range(NBUF):
        j = i * NBUF + b
        bn = (b + NBUF - 1) % NBUF
        if b == 0:

          @pl.when(j + NBUF - 1 < n_chunks)
          def _():
            @pl.when(j >= 1)
            def _():
              wait_out(bn)

            start_gather(j + NBUF - 1, bn)
        else:

          @pl.when(j + NBUF - 1 < n_chunks)
          def _():
            wait_out(bn)
            start_gather(j + NBUF - 1, bn)

        wait_gather(b)
        start_out(j, b)
      return carry

    lax.fori_loop(0, n_chunks // NBUF, body, 0)
    for b in range(NBUF):
      wait_out(b)

  return gather_kernel


@jax.jit
def kernel(tokens, table):
  b, s = tokens.shape
  vocab, d = table.shape
  dpad = 2 * d
  n = b * s
  idx = tokens.reshape(n)
  tableT = table.T
  table_p = _tc_stage(d, vocab, dpad)(tableT)
  out = _make_sc_gather(n, vocab, d, dpad)(idx, table_p)
  return out[:, :d].reshape(b, s, d)
